# baseline (speedup 1.0000x reference)
import functools

import jax
import jax.numpy as jnp
from jax import lax
from jax.experimental import pallas as pl
from jax.experimental.pallas import tpu as pltpu
from jax.experimental.pallas import tpu_sc as plsc

N_NODES = 10000
EMBED_DIM = 128
WORDS = EMBED_DIM // 2
N_EDGES = 320000

NC = 2
NS = 16
NW = NC * NS
EPW = N_EDGES // NW
CHUNK = 80
NCHUNK = EPW // CHUNK
NGROUP = CHUNK // 16


PACK_BLOCK = 104
PACK_STEPS = 6


def _body(emb_hbm, d_hbm, out_hbm,
          idx0_v, idx1_v, r0a, r0b, r1a, r1b, out_v, tbuf_v,
          fbuf_v, pbuf_v, table_sh, sem0, sem1):
  sid = lax.axis_index("s")
  wid = sid * NC + lax.axis_index("c")
  base = wid * EPW

  pltpu.sync_copy(d_hbm.at[0, pl.ds(base, EPW)], idx0_v)
  pltpu.sync_copy(d_hbm.at[1, pl.ds(base, EPW)], idx1_v)

  def pack_rows(rowbase, nrows):
    pltpu.sync_copy(emb_hbm.at[pl.ds(rowbase, nrows)],
                    fbuf_v.at[pl.ds(0, nrows)])
    def row_body(r, carry):
      for j in range(WORDS // 16):
        a = fbuf_v[r, pl.ds(j * 32, 16)]
        b = fbuf_v[r, pl.ds(j * 32 + 16, 16)]
        w = plsc.bitcast(plsc.pack(a, b, format=plsc.PackFormat.INTERLEAVED),
                         jnp.int32)
        pbuf_v[r, pl.ds(j * 16, 16)] = w
      return carry
    lax.fori_loop(0, nrows, row_body, 0)
    pltpu.sync_copy(pbuf_v.at[pl.ds(0, nrows)],
                    table_sh.at[pl.ds(rowbase, nrows)])

  def pack_step(k, carry):
    pack_rows(sid * (PACK_BLOCK * PACK_STEPS) + k * PACK_BLOCK, PACK_BLOCK)
    return carry

  lax.fori_loop(0, PACK_STEPS, pack_step, 0)

  @pl.when(sid == 0)
  def _tail():
    pack_rows(NS * PACK_BLOCK * PACK_STEPS, N_NODES - NS * PACK_BLOCK * PACK_STEPS)

  plsc.subcore_barrier()

  bufs = ((r0a, r1a, sem0), (r0b, r1b, sem1))

  def issue(c, b):
    r0, r1, sem = bufs[b]
    off = c * CHUNK
    pltpu.async_copy(table_sh.at[idx0_v.at[pl.ds(off, CHUNK)]], r0, sem)
    pltpu.async_copy(table_sh.at[idx1_v.at[pl.ds(off, CHUNK)]], r1, sem)

  def drain(c, b):
    r0, r1, sem = bufs[b]
    off = c * CHUNK
    pltpu.make_async_copy(table_sh.at[idx0_v.at[pl.ds(off, CHUNK)]], r0, sem).wait()
    pltpu.make_async_copy(table_sh.at[idx1_v.at[pl.ds(off, CHUNK)]], r1, sem).wait()

  lane = lax.iota(jnp.int32, 16)
  sh16 = jnp.full((16,), 16, jnp.uint32)

  def unpack2(w):
    wu = plsc.bitcast(w, jnp.uint32)
    lo = plsc.bitcast(lax.shift_left(wu, sh16), jnp.float32)
    hi = plsc.bitcast(wu, jnp.float32)
    return lo, hi

  def _treesum(vs):
    while len(vs) > 1:
      vs = [a + b for a, b in zip(vs[::2], vs[1::2])]
    return vs[0]

  def compute(c, b):
    r0, r1, _ = bufs[b]

    zero16 = jnp.zeros((16,), jnp.int32)

    def group_body(g, carry):
      addr0 = (g * 16 + lane) * WORDS
      accs = [jnp.zeros((16,), jnp.float32) for _ in range(4)]
      for d in range(WORDS):
        addr = addr0 + d
        wa = plsc.load_gather(r0, [zero16, addr])
        wb = plsc.load_gather(r1, [zero16, addr])
        a_lo, a_hi = unpack2(wa)
        b_lo, b_hi = unpack2(wb)
        accs[d % 4] = accs[d % 4] + (a_lo * b_lo + a_hi * b_hi)
      out_v[pl.ds(c * CHUNK + g * 16, 16)] = _treesum(accs)
      return carry

    lax.fori_loop(0, NGROUP, group_body, 0)

  issue(0, 0)
  issue(1, 1)

  def chunk_body(i, carry):
    for b in range(2):
      c = 2 * i + b

      @pl.when(c < NCHUNK)
      def _do():
        drain(c, b)
        compute(c, b)

        @pl.when(c + 2 < NCHUNK)
        def _next():
          issue(c + 2, b)

    return carry

  lax.fori_loop(0, (NCHUNK + 1) // 2, chunk_body, 0)

  pltpu.sync_copy(out_v, out_hbm.at[pl.ds(base, EPW)])


_sc_call = functools.partial(
    pl.kernel,
    out_type=jax.ShapeDtypeStruct((N_EDGES,), jnp.float32),
    mesh=plsc.VectorSubcoreMesh(core_axis_name="c", subcore_axis_name="s"),
    compiler_params=pltpu.CompilerParams(
        needs_layout_passes=False, use_tc_tiling_on_sc=False),
    scratch_types=[
        pltpu.VMEM((EPW,), jnp.int32),
        pltpu.VMEM((EPW,), jnp.int32),
        pltpu.VMEM((CHUNK, WORDS), jnp.int32),
        pltpu.VMEM((CHUNK, WORDS), jnp.int32),
        pltpu.VMEM((CHUNK, WORDS), jnp.int32),
        pltpu.VMEM((CHUNK, WORDS), jnp.int32),
        pltpu.VMEM((EPW,), jnp.float32),
        pltpu.VMEM((256,), jnp.float32),
        pltpu.VMEM((PACK_BLOCK, EMBED_DIM), jnp.float32),
        pltpu.VMEM((PACK_BLOCK, WORDS), jnp.int32),
        pltpu.VMEM_SHARED((N_NODES, WORDS), jnp.int32),
        pltpu.SemaphoreType.DMA,
        pltpu.SemaphoreType.DMA,
    ],
)(_body)


@jax.jit
def kernel(data, emb):
  return _sc_call(emb, data)


# speedup vs baseline: 3.4410x; 3.4410x over previous
import functools

import jax
import jax.numpy as jnp
from jax import lax
from jax.experimental import pallas as pl
from jax.experimental.pallas import tpu as pltpu
from jax.experimental.pallas import tpu_sc as plsc

N_NODES = 10000
EMBED_DIM = 128
WORDS = EMBED_DIM // 2
N_EDGES = 320000

NC = 2
NS = 16
NW = NC * NS
EPW = N_EDGES // NW
CHUNK = 80
NCHUNK = EPW // CHUNK
NGROUP = CHUNK // 16


PACK_BLOCK = 104
PACK_STEPS = 6


def _body(emb_hbm, d_hbm, out_hbm,
          idx0_v, idx1_v, r0a, r0b, r1a, r1b, out_v, tbuf_v,
          fbuf_v, pbuf_v, table_sh, sem0, sem1):
  sid = lax.axis_index("s")
  wid = sid * NC + lax.axis_index("c")
  base = wid * EPW

  pltpu.sync_copy(d_hbm.at[0, pl.ds(base, EPW)], idx0_v)
  pltpu.sync_copy(d_hbm.at[1, pl.ds(base, EPW)], idx1_v)

  def pack_rows(rowbase, nrows):
    pltpu.sync_copy(emb_hbm.at[pl.ds(rowbase, nrows)],
                    fbuf_v.at[pl.ds(0, nrows)])
    def row_body(r, carry):
      for j in range(WORDS // 16):
        a = fbuf_v[r, pl.ds(j * 32, 16)]
        b = fbuf_v[r, pl.ds(j * 32 + 16, 16)]
        w = plsc.bitcast(plsc.pack(a, b, format=plsc.PackFormat.INTERLEAVED),
                         jnp.int32)
        pbuf_v[r, pl.ds(j * 16, 16)] = w
      return carry
    lax.fori_loop(0, nrows, row_body, 0)
    pltpu.sync_copy(pbuf_v.at[pl.ds(0, nrows)],
                    table_sh.at[pl.ds(rowbase, nrows)])

  def pack_step(k, carry):
    pack_rows(sid * (PACK_BLOCK * PACK_STEPS) + k * PACK_BLOCK, PACK_BLOCK)
    return carry

  lax.fori_loop(0, PACK_STEPS, pack_step, 0)

  @pl.when(sid == 0)
  def _tail():
    pack_rows(NS * PACK_BLOCK * PACK_STEPS, N_NODES - NS * PACK_BLOCK * PACK_STEPS)

  plsc.subcore_barrier()

  bufs = ((r0a, r1a, sem0), (r0b, r1b, sem1))

  def issue(c, b):
    r0, r1, sem = bufs[b]
    off = c * CHUNK
    pltpu.async_copy(table_sh.at[idx0_v.at[pl.ds(off, CHUNK)]], r0, sem)
    pltpu.async_copy(table_sh.at[idx1_v.at[pl.ds(off, CHUNK)]], r1, sem)

  def drain(c, b):
    r0, r1, sem = bufs[b]
    off = c * CHUNK
    pltpu.make_async_copy(table_sh.at[idx0_v.at[pl.ds(off, CHUNK)]], r0, sem).wait()
    pltpu.make_async_copy(table_sh.at[idx1_v.at[pl.ds(off, CHUNK)]], r1, sem).wait()

  lane = lax.iota(jnp.int32, 16)
  sh16 = jnp.full((16,), 16, jnp.uint32)

  def unpack2(w):
    wu = plsc.bitcast(w, jnp.uint32)
    lo = plsc.bitcast(lax.shift_left(wu, sh16), jnp.float32)
    hi = plsc.bitcast(wu, jnp.float32)
    return lo, hi

  def _treesum(vs):
    while len(vs) > 1:
      vs = [a + b for a, b in zip(vs[::2], vs[1::2])]
    return vs[0]

  def compute(c, b):
    r0, r1, _ = bufs[b]

    def group_body(g, carry):
      for i in range(16):
        r = g * 16 + i
        ts = []
        for j in range(WORDS // 16):
          a_lo, a_hi = unpack2(r0[r, pl.ds(j * 16, 16)])
          b_lo, b_hi = unpack2(r1[r, pl.ds(j * 16, 16)])
          ts.append(a_lo * b_lo + a_hi * b_hi)
        tbuf_v[pl.ds(i * 17, 16)] = _treesum(ts)
      acc = _treesum([plsc.load_gather(tbuf_v, [lane * 17 + l])
                      for l in range(16)])
      out_v[pl.ds(c * CHUNK + g * 16, 16)] = acc
      return carry

    lax.fori_loop(0, NGROUP, group_body, 0)

  issue(0, 0)
  issue(1, 1)

  def chunk_body(i, carry):
    for b in range(2):
      c = 2 * i + b

      @pl.when(c < NCHUNK)
      def _do():
        drain(c, b)
        compute(c, b)

        @pl.when(c + 2 < NCHUNK)
        def _next():
          issue(c + 2, b)

    return carry

  lax.fori_loop(0, (NCHUNK + 1) // 2, chunk_body, 0)

  pltpu.sync_copy(out_v, out_hbm.at[pl.ds(base, EPW)])


_sc_call = functools.partial(
    pl.kernel,
    out_type=jax.ShapeDtypeStruct((N_EDGES,), jnp.float32),
    mesh=plsc.VectorSubcoreMesh(core_axis_name="c", subcore_axis_name="s"),
    compiler_params=pltpu.CompilerParams(
        needs_layout_passes=False, use_tc_tiling_on_sc=False),
    scratch_types=[
        pltpu.VMEM((EPW,), jnp.int32),
        pltpu.VMEM((EPW,), jnp.int32),
        pltpu.VMEM((CHUNK, WORDS), jnp.int32),
        pltpu.VMEM((CHUNK, WORDS), jnp.int32),
        pltpu.VMEM((CHUNK, WORDS), jnp.int32),
        pltpu.VMEM((CHUNK, WORDS), jnp.int32),
        pltpu.VMEM((EPW,), jnp.float32),
        pltpu.VMEM((272,), jnp.float32),
        pltpu.VMEM((PACK_BLOCK, EMBED_DIM), jnp.float32),
        pltpu.VMEM((PACK_BLOCK, WORDS), jnp.int32),
        pltpu.VMEM_SHARED((N_NODES, WORDS), jnp.int32),
        pltpu.SemaphoreType.DMA,
        pltpu.SemaphoreType.DMA,
    ],
)(_body)


@jax.jit
def kernel(data, emb):
  return _sc_call(emb, data)
